# Initial kernel scaffold; baseline (speedup 1.0000x reference)
import jax
import jax.numpy as jnp
from jax.experimental import pallas as pl


def kernel(x, keys, values_down, values_up, hasher_w, scale):
    raise NotImplementedError("write your pallas kernel here")


# speedup vs baseline: 8.1287x; 8.1287x over previous
import functools

import jax
import jax.numpy as jnp
from jax import lax
from jax.experimental import pallas as pl
from jax.experimental.pallas import tpu as pltpu

TOPK = 4
NEG = -1e30


def _moe_body(x_ref, keys_ref, keys_blk_ref, vdt_ref, vur_ref, hasher_ref,
              scale_ref, out_ref, h_s, m1_s, m4_s, z_s, *, n_eblk, pd):
    e = pl.program_id(1)

    @pl.when(e == 0)
    def _gates():
        h = lax.dot_general(x_ref[...], hasher_ref[...],
                            (((1,), (1,)), ((), ())),
                            preferred_element_type=jnp.float32)
        h_s[...] = h
        sim = lax.dot_general(h, keys_ref[...],
                              (((1,), (1,)), ((), ())),
                              preferred_element_type=jnp.float32)
        s = sim
        ms = []
        for _ in range(TOPK):
            m = jnp.max(s, axis=1, keepdims=True)
            ms.append(m)
            s = jnp.where(s == m, NEG, s)
        m1, m4 = ms[0], ms[TOPK - 1]
        z = sum(jnp.exp(m - m1) for m in ms)
        m1_s[...] = m1
        m4_s[...] = m4
        z_s[...] = z

    sim_blk = lax.dot_general(h_s[...], keys_blk_ref[...],
                              (((1,), (1,)), ((), ())),
                              preferred_element_type=jnp.float32)
    g_blk = jnp.where(sim_blk >= m4_s[...],
                      jnp.exp(sim_blk - m1_s[...]) / z_s[...],
                      0.0)
    e_blk = g_blk.shape[1]
    row = lax.broadcasted_iota(jnp.int32, (e_blk, e_blk * pd), 0)
    col = lax.broadcasted_iota(jnp.int32, (e_blk, e_blk * pd), 1)
    expand = (row == col // pd).astype(jnp.float32)
    g_exp = jnp.dot(g_blk, expand, preferred_element_type=jnp.float32)

    hidden = jnp.dot(x_ref[...], vdt_ref[...],
                     preferred_element_type=jnp.float32)
    act = hidden * (1.0 / (1.0 + jnp.exp(-hidden)))
    contrib = jnp.dot(g_exp * act, vur_ref[...],
                      preferred_element_type=jnp.float32)
    scale = scale_ref[0, 0]

    @pl.when(e == 0)
    def _init():
        out_ref[...] = x_ref[...] + scale * contrib

    @pl.when(e != 0)
    def _acc():
        out_ref[...] += scale * contrib


def kernel(x, keys, values_down, values_up, hasher_w, scale):
    b, t, d = x.shape
    p, pd = keys.shape
    x2 = x.reshape(t, d)
    vdt = values_down.transpose(1, 0, 2).reshape(d, p * pd)
    vur = values_up.reshape(p * pd, d)
    scale_arr = jnp.reshape(scale, (1, 1))

    n_eblk = 8
    e_blk = p // n_eblk
    t_blk = min(1024, t)
    n_tblk = t // t_blk

    grid = (n_tblk, n_eblk)
    out = pl.pallas_call(
        functools.partial(_moe_body, n_eblk=n_eblk, pd=pd),
        grid=grid,
        in_specs=[
            pl.BlockSpec((t_blk, d), lambda ti, ei: (ti, 0)),
            pl.BlockSpec((p, pd), lambda ti, ei: (0, 0)),
            pl.BlockSpec((e_blk, pd), lambda ti, ei: (ei, 0)),
            pl.BlockSpec((d, e_blk * pd), lambda ti, ei: (0, ei)),
            pl.BlockSpec((e_blk * pd, d), lambda ti, ei: (ei, 0)),
            pl.BlockSpec((pd, d), lambda ti, ei: (0, 0)),
            pl.BlockSpec(memory_space=pltpu.SMEM),
        ],
        out_specs=pl.BlockSpec((t_blk, d), lambda ti, ei: (ti, 0)),
        out_shape=jax.ShapeDtypeStruct((t, d), jnp.float32),
        scratch_shapes=[
            pltpu.VMEM((t_blk, pd), jnp.float32),
            pltpu.VMEM((t_blk, 1), jnp.float32),
            pltpu.VMEM((t_blk, 1), jnp.float32),
            pltpu.VMEM((t_blk, 1), jnp.float32),
        ],
    )(x2, keys, keys, vdt, vur, hasher_w, scale_arr)
    return out.reshape(b, t, d)
